# baseline (speedup 1.0000x reference)
import functools
import jax
import jax.numpy as jnp
from jax import lax
from jax.experimental import pallas as pl
from jax.experimental.pallas import tpu as pltpu
from jax.experimental.pallas import tpu_sc as plsc

N_ATOMS = 10000
N_NEI = 32
TBL = 4 * N_ATOMS
DEAD = TBL - 1
NR = 10240
RPW = NR // 32
CHUNK = RPW * 128
FLAT = NR * 128
ACC = 40960
DRN = ACC // 16
BATOMS = 1024
GRID = NR // BATOMS


def _sc_mesh():
    return plsc.VectorSubcoreMesh(core_axis_name="c", subcore_axis_name="s",
                                  num_cores=2, num_subcores=16)


def _gather_body(catf, imap, nb, ii, vals, tbl, sem):
    c = lax.axis_index("c")
    s = lax.axis_index("s")
    wid = s * 2 + c
    base = wid * CHUNK

    @pl.when(s == 0)
    def _():
        pltpu.sync_copy(catf, tbl)

    pltpu.sync_copy(imap.at[pl.ds(base, CHUNK)], ii)
    plsc.subcore_barrier()
    pltpu.async_copy(tbl.at[ii], vals, sem).wait()
    pltpu.sync_copy(vals, nb.at[pl.ds(base, CHUNK)])


def _sc_gather(catf, imap):
    return pl.kernel(
        _gather_body,
        out_type=jax.ShapeDtypeStruct((FLAT,), jnp.float32),
        mesh=_sc_mesh(),
        scratch_types=[
            pltpu.VMEM((CHUNK,), jnp.int32),
            pltpu.VMEM((CHUNK,), jnp.float32),
            pltpu.VMEM_SHARED((TBL,), jnp.float32),
            pltpu.SemaphoreType.DMA,
        ],
    )(catf, imap)


def _scatter_body(fe, imap, part, ii, vals, zbuf, acc, sem):
    c = lax.axis_index("c")
    s = lax.axis_index("s")
    wid = s * 2 + c
    base = wid * CHUNK

    def _z(i, carry):
        zbuf[pl.ds(i * 16, 16)] = jnp.zeros((16,), jnp.float32)
        return carry
    lax.fori_loop(0, DRN // 16, _z, 0)
    pltpu.sync_copy(zbuf, acc.at[pl.ds(s * DRN, DRN)])
    pltpu.sync_copy(imap.at[pl.ds(base, CHUNK)], ii)
    pltpu.sync_copy(fe.at[pl.ds(base, CHUNK)], vals)
    plsc.subcore_barrier()
    pltpu.sync_copy(vals, acc.at[ii], add=True)
    plsc.subcore_barrier()
    pltpu.sync_copy(acc.at[pl.ds(s * DRN, DRN)],
                    part.at[c, pl.ds(s * DRN, DRN)])


def _sc_scatter(fe, imap):
    return pl.kernel(
        _scatter_body,
        out_type=jax.ShapeDtypeStruct((2, ACC), jnp.float32),
        mesh=_sc_mesh(),
        scratch_types=[
            pltpu.VMEM((CHUNK,), jnp.int32),
            pltpu.VMEM((CHUNK,), jnp.float32),
            pltpu.VMEM((DRN,), jnp.float32),
            pltpu.VMEM_SHARED((ACC,), jnp.float32),
            pltpu.SemaphoreType.DMA,
        ],
    )(fe, imap)


def _dense_body(nb, ebc, avg, istd,
                w0, b0, w1, b1, w2, b2, w3, b3, w4, b4, w5r, b5,
                fe, ener):
    i = pl.program_id(0)
    B = BATOMS
    dot = functools.partial(jnp.dot, precision=lax.Precision.DEFAULT,
                            preferred_element_type=jnp.float32)
    dotT = functools.partial(lax.dot_general,
                             dimension_numbers=(((1,), (1,)), ((), ())),
                             precision=lax.Precision.DEFAULT,
                             preferred_element_type=jnp.float32)

    nbr = nb[...]
    cbc = jnp.dot(nbr[:, 96:104], ebc[...],
                  precision=lax.Precision.HIGHEST,
                  preferred_element_type=jnp.float32)
    rxyz = nbr[:, 0:96] - cbc
    rx = rxyz[:, 0:32]
    ry = rxyz[:, 32:64]
    rz = rxyz[:, 64:96]
    r2 = rx * rx + ry * ry + rz * rz + 1e-6
    inv_r2 = 1.0 / r2
    r = jnp.sqrt(r2)
    inv_r = 1.0 / r

    raw = jnp.concatenate([inv_r, rx * inv_r2, ry * inv_r2, rz * inv_r2],
                          axis=1)
    sel = jnp.broadcast_to(nbr[:, 99:100] == 0.0, (B, 128))
    avg_row = jnp.where(sel, jnp.broadcast_to(avg[0:1, :], (B, 128)),
                        jnp.broadcast_to(avg[1:2, :], (B, 128)))
    istd_row = jnp.where(sel, jnp.broadcast_to(istd[0:1, :], (B, 128)),
                         jnp.broadcast_to(istd[1:2, :], (B, 128)))
    sdesc = (raw - avg_row) * istd_row

    h0 = jnp.tanh(dot(sdesc, w0[...]) + b0[...])
    h1 = jnp.tanh(dot(h0, w1[...]) + b1[...])
    h2 = jnp.tanh(dot(h1, w2[...]) + b2[...])
    h3 = jnp.tanh(dot(h2, w3[...]) + b3[...])
    h4 = jnp.tanh(dot(h3, w4[...]) + b4[...])
    atom_e = jnp.sum(h4 * w5r[...], axis=1, keepdims=True) + b5[0:1, 0:1]
    rid = lax.broadcasted_iota(jnp.int32, (B, 1), 0) + i * B
    atom_e = jnp.where(rid < N_ATOMS, atom_e, 0.0)

    @pl.when(i == 0)
    def _():
        ener[...] = jnp.zeros((1, 128), jnp.float32)
    ener[...] += jnp.broadcast_to(jnp.sum(atom_e).reshape(1, 1), (1, 128))

    d4 = (1.0 - h4 * h4) * w5r[...]
    d3 = dotT(d4, w4[...]) * (1.0 - h3 * h3)
    d2 = dotT(d3, w3[...]) * (1.0 - h2 * h2)
    d1 = dotT(d2, w2[...]) * (1.0 - h1 * h1)
    d0 = dotT(d1, w1[...]) * (1.0 - h0 * h0)
    g = dotT(d0, w0[...]) * istd_row

    g0 = g[:, 0:32]
    gx = g[:, 32:64]
    gy = g[:, 64:96]
    gz = g[:, 96:128]
    gdot = gx * rx + gy * ry + gz * rz
    common = g0 * inv_r * inv_r2 + 2.0 * gdot * inv_r2 * inv_r2
    dfx = gx * inv_r2 - rx * common
    dfy = gy * inv_r2 - ry * common
    dfz = gz * inv_r2 - rz * common

    fe[...] = jnp.concatenate(
        [-dfx, -dfy, -dfz,
         jnp.sum(dfx, axis=1, keepdims=True),
         jnp.sum(dfy, axis=1, keepdims=True),
         jnp.sum(dfz, axis=1, keepdims=True),
         jnp.zeros((B, 29), jnp.float32)], axis=1)


def _tc_dense(nb, ebc, avg, istd, ws):
    B = BATOMS
    row = lambda i: (i, 0)
    fixed = lambda i: (0, 0)
    full = lambda shape: pl.BlockSpec(shape, fixed)
    in_specs = [
        pl.BlockSpec((B, 128), row),
        full((8, 96)),
        full((8, 128)),
        full((8, 128)),
    ] + [full(w.shape) for w in ws]
    out_specs = [
        pl.BlockSpec((B, 128), row),
        pl.BlockSpec((1, 128), fixed),
    ]
    out_shape = [
        jax.ShapeDtypeStruct((NR, 128), jnp.float32),
        jax.ShapeDtypeStruct((1, 128), jnp.float32),
    ]
    return pl.pallas_call(
        _dense_body,
        grid=(GRID,),
        in_specs=in_specs,
        out_specs=out_specs,
        out_shape=out_shape,
        compiler_params=pltpu.CompilerParams(
            dimension_semantics=("arbitrary",)),
    )(nb, ebc, avg, istd, *ws)


def _pad2(a, rows, cols):
    return jnp.pad(a, ((0, rows - a.shape[0]), (0, cols - a.shape[1])))


def _group_cols(t):
    return jnp.concatenate([t[:, 0::4], t[:, 1::4], t[:, 2::4], t[:, 3::4]],
                           axis=1)


def kernel(coord, atype, nlist, t_avg, t_std,
           W0, b0, W1, b1, W2, b2, W3, b3, W4, b4, W5, b5):
    catf = jnp.concatenate([coord.reshape(-1),
                            atype.reshape(-1).astype(jnp.float32)])
    nl3 = nlist[0] * 3
    ar = jnp.arange(N_ATOMS, dtype=jnp.int32)[:, None]
    imap = jnp.concatenate(
        [nl3, nl3 + 1, nl3 + 2,
         3 * ar, 3 * ar + 1, 3 * ar + 2, 3 * N_ATOMS + ar,
         jnp.full((N_ATOMS, 28), DEAD, jnp.int32)], axis=1)
    imap = jnp.pad(imap, ((0, NR - N_ATOMS), (0, 0)), constant_values=DEAD)
    imap = imap.reshape(-1)

    nb = _sc_gather(catf, imap).reshape(NR, 128)

    avg = jnp.pad(_group_cols(t_avg), ((0, 6), (0, 0)))
    istd = jnp.pad(_group_cols(1.0 / t_std), ((0, 6), (0, 0)))
    w0g = W0.reshape(32, 4, 240).transpose(1, 0, 2).reshape(128, 240)
    w0 = _pad2(w0g, 128, 256)
    ws = [w0, _pad2(b0[None, :], 1, 256),
          _pad2(W1, 256, 128), _pad2(b1[None, :], 1, 128),
          _pad2(W2, 128, 64), _pad2(b2[None, :], 1, 64),
          _pad2(W3, 64, 32), _pad2(b3[None, :], 1, 32),
          _pad2(W4, 32, 16), _pad2(b4[None, :], 1, 16),
          _pad2(W5.T, 1, 16), _pad2(b5[None, :], 1, 8)]

    ebc = jnp.pad(jnp.kron(jnp.eye(3, dtype=jnp.float32),
                           jnp.ones((1, 32), jnp.float32)), ((0, 5), (0, 0)))
    fe, ener = _tc_dense(nb, ebc, avg, istd, ws)

    part = _sc_scatter(fe.reshape(-1), imap)
    force = (part[0, :3 * N_ATOMS] + part[1, :3 * N_ATOMS]).reshape(
        1, N_ATOMS, 3)
    return ener[0, 0:1], force


# speedup vs baseline: 2.2883x; 2.2883x over previous
import functools
import jax
import jax.numpy as jnp
from jax import lax
from jax.experimental import pallas as pl
from jax.experimental.pallas import tpu as pltpu
from jax.experimental.pallas import tpu_sc as plsc

N_ATOMS = 10000
N_NEI = 32
TBL = 4 * N_ATOMS
DEAD = TBL - 1
NR = 10240
RPW = NR // 32
CHUNK = RPW * 128
FLAT = NR * 128
ACC = 40960
DRN = ACC // 16
BATOMS = 1024
GRID = NR // BATOMS


def _sc_mesh():
    return plsc.VectorSubcoreMesh(core_axis_name="c", subcore_axis_name="s",
                                  num_cores=2, num_subcores=16)


def _gather_body(catf, imap, nb, ii, vals, tbl, sem):
    c = lax.axis_index("c")
    s = lax.axis_index("s")
    wid = s * 2 + c
    base = wid * CHUNK

    @pl.when(s == 0)
    def _():
        pltpu.sync_copy(catf, tbl)

    pltpu.sync_copy(imap.at[pl.ds(base, CHUNK)], ii)
    plsc.subcore_barrier()
    pltpu.async_copy(tbl.at[ii], vals, sem).wait()
    pltpu.sync_copy(vals, nb.at[pl.ds(base, CHUNK)])


def _sc_gather(catf, imap):
    return pl.kernel(
        _gather_body,
        out_type=jax.ShapeDtypeStruct((FLAT,), jnp.float32),
        mesh=_sc_mesh(),
        scratch_types=[
            pltpu.VMEM((CHUNK,), jnp.int32),
            pltpu.VMEM((CHUNK,), jnp.float32),
            pltpu.VMEM_SHARED((TBL,), jnp.float32),
            pltpu.SemaphoreType.DMA,
        ],
    )(catf, imap)


def _scatter_body(fe, imap, part, ii, vals, zbuf, acc, sem):
    c = lax.axis_index("c")
    s = lax.axis_index("s")
    wid = s * 2 + c
    base = wid * CHUNK

    def _z(i, carry):
        zbuf[pl.ds(i * 16, 16)] = jnp.zeros((16,), jnp.float32)
        return carry
    lax.fori_loop(0, DRN // 16, _z, 0)
    pltpu.sync_copy(zbuf, acc.at[pl.ds(s * DRN, DRN)])
    pltpu.sync_copy(imap.at[pl.ds(base, CHUNK)], ii)
    pltpu.sync_copy(fe.at[pl.ds(base, CHUNK)], vals)
    plsc.subcore_barrier()
    pltpu.sync_copy(vals, acc.at[ii], add=True)
    plsc.subcore_barrier()
    pltpu.sync_copy(acc.at[pl.ds(s * DRN, DRN)],
                    part.at[c, pl.ds(s * DRN, DRN)])


def _sc_scatter(fe, imap):
    return pl.kernel(
        _scatter_body,
        out_type=jax.ShapeDtypeStruct((2, ACC), jnp.float32),
        mesh=_sc_mesh(),
        scratch_types=[
            pltpu.VMEM((CHUNK,), jnp.int32),
            pltpu.VMEM((CHUNK,), jnp.float32),
            pltpu.VMEM((DRN,), jnp.float32),
            pltpu.VMEM_SHARED((ACC,), jnp.float32),
            pltpu.SemaphoreType.DMA,
        ],
    )(fe, imap)


def _dense_body(nb, ebc, avg, istd,
                w0, b0, w1, b1, w2, b2, w3, b3, w4, b4, w5r, b5,
                fe, ener):
    i = pl.program_id(0)
    B = BATOMS
    dot = functools.partial(jnp.dot, precision=lax.Precision.DEFAULT,
                            preferred_element_type=jnp.float32)
    dotT = functools.partial(lax.dot_general,
                             dimension_numbers=(((1,), (1,)), ((), ())),
                             precision=lax.Precision.DEFAULT,
                             preferred_element_type=jnp.float32)

    nbr = nb[...]
    cbc = jnp.dot(nbr[:, 96:104], ebc[...],
                  precision=lax.Precision.HIGHEST,
                  preferred_element_type=jnp.float32)
    rxyz = nbr[:, 0:96] - cbc
    rx = rxyz[:, 0:32]
    ry = rxyz[:, 32:64]
    rz = rxyz[:, 64:96]
    r2 = rx * rx + ry * ry + rz * rz + 1e-6
    inv_r2 = 1.0 / r2
    r = jnp.sqrt(r2)
    inv_r = 1.0 / r

    raw = jnp.concatenate([inv_r, rx * inv_r2, ry * inv_r2, rz * inv_r2],
                          axis=1)
    sel = jnp.broadcast_to(nbr[:, 99:100] == 0.0, (B, 128))
    avg_row = jnp.where(sel, jnp.broadcast_to(avg[0:1, :], (B, 128)),
                        jnp.broadcast_to(avg[1:2, :], (B, 128)))
    istd_row = jnp.where(sel, jnp.broadcast_to(istd[0:1, :], (B, 128)),
                         jnp.broadcast_to(istd[1:2, :], (B, 128)))
    sdesc = (raw - avg_row) * istd_row

    h0 = jnp.tanh(dot(sdesc, w0[...]) + b0[...])
    h1 = jnp.tanh(dot(h0, w1[...]) + b1[...])
    h2 = jnp.tanh(dot(h1, w2[...]) + b2[...])
    h3 = jnp.tanh(dot(h2, w3[...]) + b3[...])
    h4 = jnp.tanh(dot(h3, w4[...]) + b4[...])
    atom_e = jnp.sum(h4 * w5r[...], axis=1, keepdims=True) + b5[0:1, 0:1]
    rid = lax.broadcasted_iota(jnp.int32, (B, 1), 0) + i * B
    atom_e = jnp.where(rid < N_ATOMS, atom_e, 0.0)

    @pl.when(i == 0)
    def _():
        ener[...] = jnp.zeros((1, 128), jnp.float32)
    ener[...] += jnp.broadcast_to(jnp.sum(atom_e).reshape(1, 1), (1, 128))

    d4 = (1.0 - h4 * h4) * w5r[...]
    d3 = dotT(d4, w4[...]) * (1.0 - h3 * h3)
    d2 = dotT(d3, w3[...]) * (1.0 - h2 * h2)
    d1 = dotT(d2, w2[...]) * (1.0 - h1 * h1)
    d0 = dotT(d1, w1[...]) * (1.0 - h0 * h0)
    g = dotT(d0, w0[...]) * istd_row

    g0 = g[:, 0:32]
    gx = g[:, 32:64]
    gy = g[:, 64:96]
    gz = g[:, 96:128]
    gdot = gx * rx + gy * ry + gz * rz
    common = g0 * inv_r * inv_r2 + 2.0 * gdot * inv_r2 * inv_r2
    dfx = gx * inv_r2 - rx * common
    dfy = gy * inv_r2 - ry * common
    dfz = gz * inv_r2 - rz * common

    fe[...] = jnp.concatenate(
        [-dfx, -dfy, -dfz,
         jnp.sum(dfx, axis=1, keepdims=True),
         jnp.sum(dfy, axis=1, keepdims=True),
         jnp.sum(dfz, axis=1, keepdims=True),
         jnp.zeros((B, 29), jnp.float32)], axis=1)


def _tc_dense(nb, ebc, avg, istd, ws):
    B = BATOMS
    row = lambda i: (i, 0)
    fixed = lambda i: (0, 0)
    full = lambda shape: pl.BlockSpec(shape, fixed)
    in_specs = [
        pl.BlockSpec((B, 128), row),
        full((8, 96)),
        full((8, 128)),
        full((8, 128)),
    ] + [full(w.shape) for w in ws]
    out_specs = [
        pl.BlockSpec((B, 128), row),
        pl.BlockSpec((1, 128), fixed),
    ]
    out_shape = [
        jax.ShapeDtypeStruct((NR, 128), jnp.float32),
        jax.ShapeDtypeStruct((1, 128), jnp.float32),
    ]
    return pl.pallas_call(
        _dense_body,
        grid=(GRID,),
        in_specs=in_specs,
        out_specs=out_specs,
        out_shape=out_shape,
        compiler_params=pltpu.CompilerParams(
            dimension_semantics=("arbitrary",)),
    )(nb, ebc, avg, istd, *ws)


def _pad2(a, rows, cols):
    return jnp.pad(a, ((0, rows - a.shape[0]), (0, cols - a.shape[1])))


def _group_cols(t):
    return jnp.concatenate([t[:, 0::4], t[:, 1::4], t[:, 2::4], t[:, 3::4]],
                           axis=1)


def kernel(coord, atype, nlist, t_avg, t_std,
           W0, b0, W1, b1, W2, b2, W3, b3, W4, b4, W5, b5):
    catf = jnp.concatenate([coord.reshape(-1),
                            atype.reshape(-1).astype(jnp.float32)])
    nl3 = nlist[0] * 3
    ar = jnp.arange(N_ATOMS, dtype=jnp.int32)[:, None]
    dl = jnp.arange(28, dtype=jnp.int32)[None, :]
    dead_sp = 3 * N_ATOMS + (ar * 28 + dl) % N_ATOMS
    imap = jnp.concatenate(
        [nl3, nl3 + 1, nl3 + 2,
         3 * ar, 3 * ar + 1, 3 * ar + 2, 3 * N_ATOMS + ar,
         dead_sp], axis=1)
    arp = jnp.arange(NR - N_ATOMS, dtype=jnp.int32)[:, None]
    lp = jnp.arange(128, dtype=jnp.int32)[None, :]
    pad_sp = 3 * N_ATOMS + (arp * 128 + lp) % N_ATOMS
    imap = jnp.concatenate([imap, pad_sp], axis=0).reshape(-1)

    nb = _sc_gather(catf, imap).reshape(NR, 128)

    avg = jnp.pad(_group_cols(t_avg), ((0, 6), (0, 0)))
    istd = jnp.pad(_group_cols(1.0 / t_std), ((0, 6), (0, 0)))
    w0g = W0.reshape(32, 4, 240).transpose(1, 0, 2).reshape(128, 240)
    w0 = _pad2(w0g, 128, 256)
    ws = [w0, _pad2(b0[None, :], 1, 256),
          _pad2(W1, 256, 128), _pad2(b1[None, :], 1, 128),
          _pad2(W2, 128, 64), _pad2(b2[None, :], 1, 64),
          _pad2(W3, 64, 32), _pad2(b3[None, :], 1, 32),
          _pad2(W4, 32, 16), _pad2(b4[None, :], 1, 16),
          _pad2(W5.T, 1, 16), _pad2(b5[None, :], 1, 8)]

    ebc = jnp.pad(jnp.kron(jnp.eye(3, dtype=jnp.float32),
                           jnp.ones((1, 32), jnp.float32)), ((0, 5), (0, 0)))
    fe, ener = _tc_dense(nb, ebc, avg, istd, ws)

    part = _sc_scatter(fe.reshape(-1), imap)
    force = (part[0, :3 * N_ATOMS] + part[1, :3 * N_ATOMS]).reshape(
        1, N_ATOMS, 3)
    return ener[0, 0:1], force


# speedup vs baseline: 2.3496x; 1.0268x over previous
import functools
import jax
import jax.numpy as jnp
from jax import lax
from jax.experimental import pallas as pl
from jax.experimental.pallas import tpu as pltpu
from jax.experimental.pallas import tpu_sc as plsc

N_ATOMS = 10000
N_NEI = 32
TBL = 4 * N_ATOMS
DEAD = TBL - 1
NR = 10240
RPW = NR // 32
CHUNK = RPW * 128
FLAT = NR * 128
ACC = 40960
DRN = ACC // 16
BATOMS = 2048
GRID = NR // BATOMS


def _sc_mesh():
    return plsc.VectorSubcoreMesh(core_axis_name="c", subcore_axis_name="s",
                                  num_cores=2, num_subcores=16)


def _gather_body(catf, imap, nb, ii, vals, tbl, sem):
    c = lax.axis_index("c")
    s = lax.axis_index("s")
    wid = s * 2 + c
    base = wid * CHUNK

    @pl.when(s == 0)
    def _():
        pltpu.sync_copy(catf, tbl)

    pltpu.sync_copy(imap.at[pl.ds(base, CHUNK)], ii)
    plsc.subcore_barrier()
    pltpu.async_copy(tbl.at[ii], vals, sem).wait()
    pltpu.sync_copy(vals, nb.at[pl.ds(base, CHUNK)])


def _sc_gather(catf, imap):
    return pl.kernel(
        _gather_body,
        out_type=jax.ShapeDtypeStruct((FLAT,), jnp.float32),
        mesh=_sc_mesh(),
        scratch_types=[
            pltpu.VMEM((CHUNK,), jnp.int32),
            pltpu.VMEM((CHUNK,), jnp.float32),
            pltpu.VMEM_SHARED((TBL,), jnp.float32),
            pltpu.SemaphoreType.DMA,
        ],
    )(catf, imap)


def _scatter_body(fe, imap, part, ii, vals, zbuf, acc, sem):
    c = lax.axis_index("c")
    s = lax.axis_index("s")
    wid = s * 2 + c
    base = wid * CHUNK

    def _z(i, carry):
        zbuf[pl.ds(i * 16, 16)] = jnp.zeros((16,), jnp.float32)
        return carry
    lax.fori_loop(0, DRN // 16, _z, 0)
    pltpu.sync_copy(zbuf, acc.at[pl.ds(s * DRN, DRN)])
    pltpu.sync_copy(imap.at[pl.ds(base, CHUNK)], ii)
    pltpu.sync_copy(fe.at[pl.ds(base, CHUNK)], vals)
    plsc.subcore_barrier()
    pltpu.sync_copy(vals, acc.at[ii], add=True)
    plsc.subcore_barrier()
    pltpu.sync_copy(acc.at[pl.ds(s * DRN, DRN)],
                    part.at[c, pl.ds(s * DRN, DRN)])


def _sc_scatter(fe, imap):
    return pl.kernel(
        _scatter_body,
        out_type=jax.ShapeDtypeStruct((2, ACC), jnp.float32),
        mesh=_sc_mesh(),
        scratch_types=[
            pltpu.VMEM((CHUNK,), jnp.int32),
            pltpu.VMEM((CHUNK,), jnp.float32),
            pltpu.VMEM((DRN,), jnp.float32),
            pltpu.VMEM_SHARED((ACC,), jnp.float32),
            pltpu.SemaphoreType.DMA,
        ],
    )(fe, imap)


def _dense_body(nb, ebc, avg, istd,
                w0, b0, w1, b1, w2, b2, w3, b3, w4, b4, w5r, b5,
                fe, ener):
    i = pl.program_id(0)
    B = BATOMS
    dot = functools.partial(jnp.dot, precision=lax.Precision.DEFAULT,
                            preferred_element_type=jnp.float32)
    dotT = functools.partial(lax.dot_general,
                             dimension_numbers=(((1,), (1,)), ((), ())),
                             precision=lax.Precision.DEFAULT,
                             preferred_element_type=jnp.float32)

    nbr = nb[...]
    cbc = jnp.dot(nbr[:, 96:104], ebc[...],
                  precision=lax.Precision.HIGHEST,
                  preferred_element_type=jnp.float32)
    rxyz = nbr[:, 0:96] - cbc
    rx = rxyz[:, 0:32]
    ry = rxyz[:, 32:64]
    rz = rxyz[:, 64:96]
    r2 = rx * rx + ry * ry + rz * rz + 1e-6
    inv_r2 = 1.0 / r2
    r = jnp.sqrt(r2)
    inv_r = 1.0 / r

    raw = jnp.concatenate([inv_r, rx * inv_r2, ry * inv_r2, rz * inv_r2],
                          axis=1)
    sel = jnp.broadcast_to(nbr[:, 99:100] == 0.0, (B, 128))
    avg_row = jnp.where(sel, jnp.broadcast_to(avg[0:1, :], (B, 128)),
                        jnp.broadcast_to(avg[1:2, :], (B, 128)))
    istd_row = jnp.where(sel, jnp.broadcast_to(istd[0:1, :], (B, 128)),
                         jnp.broadcast_to(istd[1:2, :], (B, 128)))
    sdesc = (raw - avg_row) * istd_row

    h0 = jnp.tanh(dot(sdesc, w0[...]) + b0[...])
    h1 = jnp.tanh(dot(h0, w1[...]) + b1[...])
    h2 = jnp.tanh(dot(h1, w2[...]) + b2[...])
    h3 = jnp.tanh(dot(h2, w3[...]) + b3[...])
    h4 = jnp.tanh(dot(h3, w4[...]) + b4[...])
    atom_e = jnp.sum(h4 * w5r[...], axis=1, keepdims=True) + b5[0:1, 0:1]
    rid = lax.broadcasted_iota(jnp.int32, (B, 1), 0) + i * B
    atom_e = jnp.where(rid < N_ATOMS, atom_e, 0.0)

    @pl.when(i == 0)
    def _():
        ener[...] = jnp.zeros((1, 128), jnp.float32)
    ener[...] += jnp.broadcast_to(jnp.sum(atom_e).reshape(1, 1), (1, 128))

    d4 = (1.0 - h4 * h4) * w5r[...]
    d3 = dotT(d4, w4[...]) * (1.0 - h3 * h3)
    d2 = dotT(d3, w3[...]) * (1.0 - h2 * h2)
    d1 = dotT(d2, w2[...]) * (1.0 - h1 * h1)
    d0 = dotT(d1, w1[...]) * (1.0 - h0 * h0)
    g = dotT(d0, w0[...]) * istd_row

    g0 = g[:, 0:32]
    gx = g[:, 32:64]
    gy = g[:, 64:96]
    gz = g[:, 96:128]
    gdot = gx * rx + gy * ry + gz * rz
    common = g0 * inv_r * inv_r2 + 2.0 * gdot * inv_r2 * inv_r2
    dfx = gx * inv_r2 - rx * common
    dfy = gy * inv_r2 - ry * common
    dfz = gz * inv_r2 - rz * common

    fe[...] = jnp.concatenate(
        [-dfx, -dfy, -dfz,
         jnp.sum(dfx, axis=1, keepdims=True),
         jnp.sum(dfy, axis=1, keepdims=True),
         jnp.sum(dfz, axis=1, keepdims=True),
         jnp.zeros((B, 29), jnp.float32)], axis=1)


def _tc_dense(nb, ebc, avg, istd, ws):
    B = BATOMS
    row = lambda i: (i, 0)
    fixed = lambda i: (0, 0)
    full = lambda shape: pl.BlockSpec(shape, fixed)
    in_specs = [
        pl.BlockSpec((B, 128), row),
        full((8, 96)),
        full((8, 128)),
        full((8, 128)),
    ] + [full(w.shape) for w in ws]
    out_specs = [
        pl.BlockSpec((B, 128), row),
        pl.BlockSpec((1, 128), fixed),
    ]
    out_shape = [
        jax.ShapeDtypeStruct((NR, 128), jnp.float32),
        jax.ShapeDtypeStruct((1, 128), jnp.float32),
    ]
    return pl.pallas_call(
        _dense_body,
        grid=(GRID,),
        in_specs=in_specs,
        out_specs=out_specs,
        out_shape=out_shape,
        compiler_params=pltpu.CompilerParams(
            dimension_semantics=("arbitrary",)),
    )(nb, ebc, avg, istd, *ws)


def _pad2(a, rows, cols):
    return jnp.pad(a, ((0, rows - a.shape[0]), (0, cols - a.shape[1])))


def _group_cols(t):
    return jnp.concatenate([t[:, 0::4], t[:, 1::4], t[:, 2::4], t[:, 3::4]],
                           axis=1)


def kernel(coord, atype, nlist, t_avg, t_std,
           W0, b0, W1, b1, W2, b2, W3, b3, W4, b4, W5, b5):
    catf = jnp.concatenate([coord.reshape(-1),
                            atype.reshape(-1).astype(jnp.float32)])
    nl3 = nlist[0] * 3
    ar = jnp.arange(N_ATOMS, dtype=jnp.int32)[:, None]
    dl = jnp.arange(28, dtype=jnp.int32)[None, :]
    dead_sp = 3 * N_ATOMS + (ar * 28 + dl) % N_ATOMS
    imap = jnp.concatenate(
        [nl3, nl3 + 1, nl3 + 2,
         3 * ar, 3 * ar + 1, 3 * ar + 2, 3 * N_ATOMS + ar,
         dead_sp], axis=1)
    arp = jnp.arange(NR - N_ATOMS, dtype=jnp.int32)[:, None]
    lp = jnp.arange(128, dtype=jnp.int32)[None, :]
    pad_sp = 3 * N_ATOMS + (arp * 128 + lp) % N_ATOMS
    imap = jnp.concatenate([imap, pad_sp], axis=0).reshape(-1)

    nb = _sc_gather(catf, imap).reshape(NR, 128)

    avg = jnp.pad(_group_cols(t_avg), ((0, 6), (0, 0)))
    istd = jnp.pad(_group_cols(1.0 / t_std), ((0, 6), (0, 0)))
    w0g = W0.reshape(32, 4, 240).transpose(1, 0, 2).reshape(128, 240)
    w0 = _pad2(w0g, 128, 256)
    ws = [w0, _pad2(b0[None, :], 1, 256),
          _pad2(W1, 256, 128), _pad2(b1[None, :], 1, 128),
          _pad2(W2, 128, 64), _pad2(b2[None, :], 1, 64),
          _pad2(W3, 64, 32), _pad2(b3[None, :], 1, 32),
          _pad2(W4, 32, 16), _pad2(b4[None, :], 1, 16),
          _pad2(W5.T, 1, 16), _pad2(b5[None, :], 1, 8)]

    ebc = jnp.pad(jnp.kron(jnp.eye(3, dtype=jnp.float32),
                           jnp.ones((1, 32), jnp.float32)), ((0, 5), (0, 0)))
    fe, ener = _tc_dense(nb, ebc, avg, istd, ws)

    part = _sc_scatter(fe.reshape(-1), imap)
    force = (part[0, :3 * N_ATOMS] + part[1, :3 * N_ATOMS]).reshape(
        1, N_ATOMS, 3)
    return ener[0, 0:1], force
